# baseline (speedup 1.0000x reference)
import functools
import jax
import jax.numpy as jnp
from jax import lax
from jax.experimental import pallas as pl
from jax.experimental.pallas import tpu as pltpu
from jax.experimental.pallas import tpu_sc as plsc

_B = 16384
_E = 64
_NC = 2
_NS = 16
_NW = _NC * _NS
_BPW = _B // _NW
_PH = 256
_CH = 128


def _make_kernel():
    mesh = plsc.VectorSubcoreMesh(core_axis_name="c", subcore_axis_name="s")

    @functools.partial(
        pl.kernel,
        mesh=mesh,
        out_type=jax.ShapeDtypeStruct((_B,), jnp.float32),
        compiler_params=pltpu.CompilerParams(
            needs_layout_passes=False,
            skip_device_barrier=True,
        ),
        scratch_types=[
            pltpu.VMEM((4, _CH), jnp.int32),
            pltpu.VMEM((4, _CH), jnp.int32),
            pltpu.VMEM((4, _CH), jnp.int32),
            pltpu.VMEM((4, _CH), jnp.int32),
            pltpu.VMEM((4, _CH), jnp.int32),
            pltpu.VMEM((4, _CH), jnp.int32),
            pltpu.VMEM((_PH, 128), jnp.float32),
            pltpu.VMEM((_PH, 128), jnp.float32),
            pltpu.VMEM((_PH, 128), jnp.float32),
            pltpu.VMEM((_BPW,), jnp.float32),
            pltpu.SemaphoreType.DMA,
        ],
    )
    def lp_kernel(s_hbm, p_hbm, o_hbm, ent2_hbm, rel2_hbm, out_hbm,
                  s_idx, p_idx, o_idx, s_pair, p_pair, o_pair,
                  s_rows, p_rows, o_rows, out_v, sem):
        wid = lax.axis_index("s") * _NC + lax.axis_index("c")
        base = wid * _BPW

        for j in range(4):
            pltpu.sync_copy(s_hbm.at[pl.ds(base + j * _CH, _CH)], s_idx.at[j])
            pltpu.sync_copy(p_hbm.at[pl.ds(base + j * _CH, _CH)], p_idx.at[j])
            pltpu.sync_copy(o_hbm.at[pl.ds(base + j * _CH, _CH)], o_idx.at[j])

        for j in range(4):
            for k in range(8):
                sl = pl.ds(k * 16, 16)
                s_pair[j, sl] = lax.shift_right_logical(s_idx[j, sl], 1)
                p_pair[j, sl] = lax.shift_right_logical(p_idx[j, sl], 1)
                o_pair[j, sl] = lax.shift_right_logical(o_idx[j, sl], 1)

        lane = lax.iota(jnp.int32, 16)

        for ph in range(2):
            copies = []
            for j in range(2):
                rows = pl.ds(j * _CH, _CH)
                jj = ph * 2 + j
                copies.append(pltpu.async_copy(ent2_hbm.at[s_pair.at[jj]], s_rows.at[rows], sem))
                copies.append(pltpu.async_copy(rel2_hbm.at[p_pair.at[jj]], p_rows.at[rows], sem))
                copies.append(pltpu.async_copy(ent2_hbm.at[o_pair.at[jj]], o_rows.at[rows], sem))
            for c in copies:
                c.wait()

            def chunk_body(ci, carry):
                row_ids = ci * 16 + lane
                jj = ph * 2 + ci // 8
                kk = ci % 8
                sl = pl.ds(kk * 16, 16)
                s_par = lax.bitwise_and(s_idx[jj, sl], 1) * _E
                p_par = lax.bitwise_and(p_idx[jj, sl], 1) * _E
                o_par = lax.bitwise_and(o_idx[jj, sl], 1) * _E
                acc = jnp.zeros((16,), jnp.float32)
                for e in range(_E):
                    a = plsc.load_gather(s_rows, [row_ids, s_par + e])
                    b = plsc.load_gather(p_rows, [row_ids, p_par + e])
                    c = plsc.load_gather(o_rows, [row_ids, o_par + e])
                    acc = acc + a * b * c
                out_v[pl.ds(ph * _PH + ci * 16, 16)] = acc
                return carry

            lax.fori_loop(0, _PH // 16, chunk_body, 0)

        pltpu.sync_copy(out_v, out_hbm.at[pl.ds(base, _BPW)])

    return lp_kernel


_lp_kernel = None


def kernel(s, p, o, entities, relations):
    global _lp_kernel
    if _lp_kernel is None:
        _lp_kernel = _make_kernel()
    ent2 = jnp.reshape(entities, (entities.shape[0] // 2, 2 * entities.shape[1]))
    rel2 = jnp.reshape(relations, (relations.shape[0] // 2, 2 * relations.shape[1]))
    return _lp_kernel(s, p, o, ent2, rel2)


# speedup vs baseline: 1.0834x; 1.0834x over previous
import functools
import jax
import jax.numpy as jnp
from jax import lax
from jax.experimental import pallas as pl
from jax.experimental.pallas import tpu as pltpu
from jax.experimental.pallas import tpu_sc as plsc

_B = 16384
_E = 64
_NC = 2
_NS = 16
_NW = _NC * _NS
_BPW = _B // _NW
_PH = 256
_CH = 128


def _make_kernel():
    mesh = plsc.VectorSubcoreMesh(core_axis_name="c", subcore_axis_name="s")

    @functools.partial(
        pl.kernel,
        mesh=mesh,
        out_type=jax.ShapeDtypeStruct((_B,), jnp.float32),
        compiler_params=pltpu.CompilerParams(needs_layout_passes=False),
        scratch_types=[
            pltpu.VMEM((4, _CH), jnp.int32),
            pltpu.VMEM((4, _CH), jnp.int32),
            pltpu.VMEM((4, _CH), jnp.int32),
            pltpu.VMEM((_PH, 128), jnp.float32),
            pltpu.VMEM((_PH, 128), jnp.float32),
            pltpu.VMEM((_PH, 128), jnp.float32),
            pltpu.VMEM((_BPW,), jnp.float32),
            pltpu.SemaphoreType.DMA,
        ],
    )
    def lp_kernel(s_hbm, p_hbm, o_hbm, ent_hbm, rel_hbm, out_hbm,
                  s_idx, p_idx, o_idx, s_rows, p_rows, o_rows, out_v, sem):
        wid = lax.axis_index("s") * _NC + lax.axis_index("c")
        base = wid * _BPW

        for j in range(4):
            pltpu.sync_copy(s_hbm.at[pl.ds(base + j * _CH, _CH)], s_idx.at[j])
            pltpu.sync_copy(p_hbm.at[pl.ds(base + j * _CH, _CH)], p_idx.at[j])
            pltpu.sync_copy(o_hbm.at[pl.ds(base + j * _CH, _CH)], o_idx.at[j])

        lane = lax.iota(jnp.int32, 16)

        for ph in range(2):
            copies = []
            for j in range(2):
                rows = pl.ds(j * _CH, _CH)
                jj = ph * 2 + j
                copies.append(pltpu.async_copy(ent_hbm.at[s_idx.at[jj]], s_rows.at[rows], sem))
                copies.append(pltpu.async_copy(rel_hbm.at[p_idx.at[jj]], p_rows.at[rows], sem))
                copies.append(pltpu.async_copy(ent_hbm.at[o_idx.at[jj]], o_rows.at[rows], sem))
            for c in copies:
                c.wait()

            def chunk_body(ci, carry):
                row_ids = ci * 16 + lane
                acc = jnp.zeros((16,), jnp.float32)
                for e in range(_E):
                    col = jnp.full((16,), e, dtype=jnp.int32)
                    a = plsc.load_gather(s_rows, [row_ids, col])
                    b = plsc.load_gather(p_rows, [row_ids, col])
                    c = plsc.load_gather(o_rows, [row_ids, col])
                    acc = acc + a * b * c
                out_v[pl.ds(ph * _PH + ci * 16, 16)] = acc
                return carry

            lax.fori_loop(0, _PH // 16, chunk_body, 0)

        pltpu.sync_copy(out_v, out_hbm.at[pl.ds(base, _BPW)])

    return lp_kernel


_lp_kernel = None


def kernel(s, p, o, entities, relations):
    global _lp_kernel
    if _lp_kernel is None:
        _lp_kernel = _make_kernel()
    ent_pad = jnp.pad(entities, ((0, 0), (0, 128 - entities.shape[1])))
    rel_pad = jnp.pad(relations, ((0, 0), (0, 128 - relations.shape[1])))
    return _lp_kernel(s, p, o, ent_pad, rel_pad)
